# baseline (speedup 1.0000x reference)
import functools

import jax
import jax.numpy as jnp
from jax import lax
from jax.experimental import pallas as pl
from jax.experimental.pallas import tpu as pltpu
from jax.experimental.pallas import tpu_sc as plsc

K = 16
D = K * K
L = 16
CHUNK = 4096
NR = 3
RSIZE = 33408


@functools.cache
def _make_gather(V: int, B: int):
  info = plsc.get_sparse_core_info()
  nc, ns = info.num_cores, info.num_subcores
  nw = nc * ns
  assert D % nw == 0
  d_per_w = D // nw
  n_flush = B // CHUNK
  gpc = CHUNK // L
  starts = [min(r * RSIZE, V - RSIZE) for r in range(NR)]
  assert NR * RSIZE >= V
  assert all(starts[r] == r * RSIZE for r in range(NR - 1))
  mesh = plsc.VectorSubcoreMesh(core_axis_name="c", subcore_axis_name="s")

  @functools.partial(
      pl.kernel,
      out_type=jax.ShapeDtypeStruct((D, B), jnp.float32),
      mesh=mesh,
      compiler_params=pltpu.CompilerParams(needs_layout_passes=False),
      scratch_types=[
          pltpu.VMEM((RSIZE,), jnp.float32),
          pltpu.VMEM((RSIZE,), jnp.float32),
          pltpu.VMEM((B,), jnp.int32),
          pltpu.VMEM((B,), jnp.float32),
          pltpu.SemaphoreType.DMA,
          pltpu.SemaphoreType.DMA,
          pltpu.SemaphoreType.DMA,
          pltpu.SemaphoreType.DMA,
          pltpu.SemaphoreType.DMA,
          pltpu.SemaphoreType.DMA,
      ],
  )
  def gather(table_t, idx_hbm, out_t, buf0, buf1, idx_v, row, ssem0, ssem1,
             f0, f1, f2, f3):
    wid = lax.axis_index("s") * nc + lax.axis_index("c")
    d0 = wid * d_per_w
    bufs = (buf0, buf1)
    ssems = (ssem0, ssem1)
    fsems = (f0, f1, f2, f3)
    nu = d_per_w * NR

    def stage(u):
      j, r = divmod(u, NR)
      return pltpu.async_copy(
          table_t.at[pl.ds((d0 + j) * V + starts[r], RSIZE)],
          bufs[u % 2], ssems[u % 2])

    cur = stage(0)
    pltpu.sync_copy(idx_hbm, idx_v)
    flushes = [None] * n_flush
    for j in range(d_per_w):
      d = d0 + j
      for r in range(NR):
        u = j * NR + r
        cur.wait()
        nxt = stage(u + 1) if u + 1 < nu else None
        s0 = starts[r]
        own_lo = r * RSIZE
        bp = bufs[u % 2]
        for h in range(n_flush):
          if r == 0 and flushes[h] is not None:
            flushes[h].wait()

          def body(g, h=h, r=r, s0=s0, own_lo=own_lo, bp=bp):
            sl = pl.ds(h * CHUNK + g * L, L)
            iv = idx_v[sl]
            if r == 0:
              loc = jnp.minimum(iv, RSIZE - 1)
              row[sl] = plsc.load_gather(bp, [loc])
            elif r == NR - 1:
              loc = jnp.maximum(iv - s0, 0)
              val = plsc.load_gather(bp, [loc])
              row[sl] = jnp.where(iv >= own_lo, val, row[sl])
            else:
              t = iv - s0
              loc = jnp.minimum(jnp.maximum(t, 0), RSIZE - 1)
              val = plsc.load_gather(bp, [loc])
              row[sl] = jnp.where(t == loc, val, row[sl])

          plsc.parallel_loop(0, gpc, 1, unroll=8)(body)
          if r == NR - 1:
            flushes[h] = pltpu.async_copy(
                row.at[pl.ds(h * CHUNK, CHUNK)],
                out_t.at[d, pl.ds(h * CHUNK, CHUNK)], fsems[h])
        cur = nxt
    for fl in flushes:
      fl.wait()

  return gather


def kernel(idx, frames):
  V = frames.shape[0]
  B = idx.shape[0]
  table_flat = frames.reshape(V, D).T.reshape(V * D)
  out_t = _make_gather(V, B)(table_flat, idx.astype(jnp.int32))
  return out_t.T.reshape(B, K, K)


# speedup vs baseline: 1.0015x; 1.0015x over previous
import functools

import jax
import jax.numpy as jnp
from jax import lax
from jax.experimental import pallas as pl
from jax.experimental.pallas import tpu as pltpu
from jax.experimental.pallas import tpu_sc as plsc

K = 16
D = K * K
L = 16
CHUNK = 4096
NR = 3
RSIZE = 33408


@functools.cache
def _make_gather(V: int, B: int):
  info = plsc.get_sparse_core_info()
  nc, ns = info.num_cores, info.num_subcores
  nw = nc * ns
  assert D % nw == 0
  d_per_w = D // nw
  n_flush = B // CHUNK
  gpc = CHUNK // L
  starts = [min(r * RSIZE, V - RSIZE) for r in range(NR)]
  assert NR * RSIZE >= V
  mesh = plsc.VectorSubcoreMesh(core_axis_name="c", subcore_axis_name="s")

  @functools.partial(
      pl.kernel,
      out_type=jax.ShapeDtypeStruct((D, B), jnp.float32),
      mesh=mesh,
      compiler_params=pltpu.CompilerParams(needs_layout_passes=False),
      scratch_types=[
          pltpu.VMEM((RSIZE,), jnp.float32),
          pltpu.VMEM((RSIZE,), jnp.float32),
          pltpu.VMEM((B,), jnp.int32),
          pltpu.VMEM((B,), jnp.float32),
          pltpu.SemaphoreType.DMA,
          pltpu.SemaphoreType.DMA,
          pltpu.SemaphoreType.DMA,
          pltpu.SemaphoreType.DMA,
          pltpu.SemaphoreType.DMA,
          pltpu.SemaphoreType.DMA,
      ],
  )
  def gather(table_t, idx_hbm, out_t, buf0, buf1, idx_v, row, ssem0, ssem1,
             f0, f1, f2, f3):
    wid = lax.axis_index("s") * nc + lax.axis_index("c")
    d0 = wid * d_per_w
    bufs = (buf0, buf1)
    ssems = (ssem0, ssem1)
    fsems = (f0, f1, f2, f3)
    nu = d_per_w * NR

    def stage(u):
      j, r = divmod(u, NR)
      return pltpu.async_copy(
          table_t.at[pl.ds((d0 + j) * V + starts[r], RSIZE)],
          bufs[u % 2], ssems[u % 2])

    cur = stage(0)
    pltpu.sync_copy(idx_hbm, idx_v)
    flushes = [None] * n_flush
    for j in range(d_per_w):
      d = d0 + j
      for r in range(NR):
        u = j * NR + r
        cur.wait()
        nxt = stage(u + 1) if u + 1 < nu else None
        s0 = starts[r]
        own_lo = r * RSIZE
        bp = bufs[u % 2]
        for h in range(n_flush):
          if r == 0 and flushes[h] is not None:
            flushes[h].wait()

          def body(g, h=h, r=r, s0=s0, own_lo=own_lo, bp=bp):
            sl = pl.ds(h * CHUNK + g * L, L)
            iv = idx_v[sl]
            loc = jnp.minimum(jnp.maximum(iv - s0, 0), RSIZE - 1)
            val = plsc.load_gather(bp, [loc])
            if r == 0:
              row[sl] = val
            elif r == NR - 1:
              row[sl] = jnp.where(iv >= own_lo, val, row[sl])
            else:
              row[sl] = jnp.where((iv >= own_lo) & (iv < own_lo + RSIZE),
                                  val, row[sl])

          plsc.parallel_loop(0, gpc, 1, unroll=8)(body)
          if r == NR - 1:
            flushes[h] = pltpu.async_copy(
                row.at[pl.ds(h * CHUNK, CHUNK)],
                out_t.at[d, pl.ds(h * CHUNK, CHUNK)], fsems[h])
        cur = nxt
    for fl in flushes:
      fl.wait()

  return gather


def kernel(idx, frames):
  V = frames.shape[0]
  B = idx.shape[0]
  table_flat = frames.reshape(V, D).T.reshape(V * D)
  out_t = _make_gather(V, B)(table_flat, idx.astype(jnp.int32))
  return out_t.T.reshape(B, K, K)


# speedup vs baseline: 2.9007x; 2.8962x over previous
import functools

import jax
import jax.numpy as jnp
from jax import lax
from jax.experimental import pallas as pl
from jax.experimental.pallas import tpu as pltpu
from jax.experimental.pallas import tpu_sc as plsc

K = 16
D = K * K
L = 16
OUT_CHUNK = 4096


@functools.cache
def _make_gather(V: int, B: int):
  info = plsc.get_sparse_core_info()
  nc, ns = info.num_cores, info.num_subcores
  nw = nc * ns
  assert D % nw == 0
  d_per_w = D // nw
  n_flush = B // OUT_CHUNK
  groups_per_flush = OUT_CHUNK // L
  mesh = plsc.VectorSubcoreMesh(core_axis_name="c", subcore_axis_name="s")

  @functools.partial(
      pl.kernel,
      out_type=jax.ShapeDtypeStruct((D, B), jnp.float32),
      mesh=mesh,
      compiler_params=pltpu.CompilerParams(needs_layout_passes=False),
      scratch_types=[
          pltpu.VMEM((V,), jnp.float32),
          pltpu.VMEM((B,), jnp.int32),
          pltpu.VMEM((2, OUT_CHUNK), jnp.float32),
          pltpu.SemaphoreType.DMA,
          pltpu.SemaphoreType.DMA,
          pltpu.SemaphoreType.DMA,
          pltpu.SemaphoreType.DMA,
      ],
  )
  def gather(table_t, idx_hbm, out_t, col_v, idx_v, obuf, osem0, osem1,
             csem0, csem1):
    wid = lax.axis_index("s") * nc + lax.axis_index("c")
    d0 = wid * d_per_w
    cnext = pltpu.async_copy(table_t.at[d0], col_v, csem0)
    pltpu.sync_copy(idx_hbm, idx_v)
    osems = (osem0, osem1)
    stores = [None, None]
    for j in range(d_per_w):
      d = d0 + j
      cnext.wait()
      for h in range(n_flush):
        hb = h % 2
        def body(g, h=h, hb=hb):
          iv = idx_v[pl.ds(h * OUT_CHUNK + g * L, L)]
          obuf[hb, pl.ds(g * L, L)] = plsc.load_gather(col_v, [iv])
        plsc.parallel_loop(0, groups_per_flush, 1, unroll=8)(body)
        if stores[hb] is not None:
          stores[hb].wait()
        stores[hb] = pltpu.async_copy(
            obuf.at[hb], out_t.at[d, pl.ds(h * OUT_CHUNK, OUT_CHUNK)],
            osems[hb])
      if j + 1 < d_per_w:
        cnext = pltpu.async_copy(
            table_t.at[d + 1], col_v, csem0 if j % 2 else csem1)
    for st in stores:
      if st is not None:
        st.wait()

  return gather


def kernel(idx, frames):
  V = frames.shape[0]
  B = idx.shape[0]
  table_t = frames.reshape(V, D).T
  out_t = _make_gather(V, B)(table_t, idx.astype(jnp.int32))
  return out_t.T.reshape(B, K, K)
